# baseline (speedup 1.0000x reference)
import functools
import jax
import jax.numpy as jnp
from jax.experimental import pallas as pl
from jax.experimental.pallas import tpu as pltpu

N = 10000
E = 320000
H = 128
K_TOP = 15
L_GGNN = 2

ROW_BLK = 1000


def _mm_kernel(x_ref, w_ref, o_ref):
    o_ref[...] = jnp.dot(x_ref[...], w_ref[...],
                         preferred_element_type=jnp.float32)


def _pallas_matmul(x, w):
    n, h = x.shape
    m = w.shape[1]
    grid = (n // ROW_BLK,)
    return pl.pallas_call(
        _mm_kernel,
        grid=grid,
        in_specs=[
            pl.BlockSpec((ROW_BLK, h), lambda i: (i, 0)),
            pl.BlockSpec((h, m), lambda i: (0, 0)),
        ],
        out_specs=pl.BlockSpec((ROW_BLK, m), lambda i: (i, 0)),
        out_shape=jax.ShapeDtypeStruct((n, m), jnp.float32),
    )(x, w)


def _gcn_conv(x, src, dst, dinv, W, b):
    norm = dinv[src] * dinv[dst]
    xw = _pallas_matmul(x, W)
    out = jnp.zeros((N, H), jnp.float32).at[dst].add(norm[:, None] * xw[src])
    out = out + dinv[:, None] * dinv[:, None] * xw
    return out + b


def _per_node_topk_mask(dst, scores, k):
    order = jnp.lexsort((-scores, dst))
    dst_s = dst[order]
    start = jnp.searchsorted(dst_s, dst_s, side='left')
    rank = jnp.arange(dst.shape[0]) - start
    mask_s = rank < k
    return jnp.zeros((dst.shape[0],), bool).at[order].set(mask_s)


def kernel(inputs, A, emb, W1, b1, W2, b2, ggnn_w, w_ih, w_hh, b_ih, b_hh):
    hidden = emb
    src, dst = A[0], A[1]
    deg = jnp.ones((N,), jnp.float32).at[dst].add(1.0)
    dinv = 1.0 / jnp.sqrt(deg)

    h = jax.nn.relu(_gcn_conv(hidden, src, dst, dinv, W1, b1))
    h = jax.nn.relu(_gcn_conv(h, src, dst, dinv, W2, b2))

    nrm = jnp.maximum(jnp.linalg.norm(h, axis=1), 1e-8)
    scores = jnp.sum(h[src] * h[dst], axis=1) / (nrm[src] * nrm[dst])
    mask = _per_node_topk_mask(dst, scores, K_TOP).astype(jnp.float32)

    x = hidden
    for i in range(L_GGNN):
        m = _pallas_matmul(x, ggnn_w[i])
        agg = jnp.zeros((N, H), jnp.float32).at[dst].add(mask[:, None] * m[src])
        gi = _pallas_matmul(agg, w_ih.T) + b_ih
        gh = _pallas_matmul(x, w_hh.T) + b_hh
        i_r, i_z, i_n = jnp.split(gi, 3, axis=1)
        h_r, h_z, h_n = jnp.split(gh, 3, axis=1)
        r = jax.nn.sigmoid(i_r + h_r)
        z = jax.nn.sigmoid(i_z + h_z)
        nn_ = jnp.tanh(i_n + r * h_n)
        x = (1.0 - z) * nn_ + z * x
    return x


# speedup vs baseline: 1.5906x; 1.5906x over previous
import functools
import jax
import jax.numpy as jnp
from jax import lax
from jax.experimental import pallas as pl
from jax.experimental.pallas import tpu as pltpu
from jax.experimental.pallas import tpu_sc as plsc

N = 10000
E = 320000
H = 128
K_TOP = 15
L_GGNN = 2

_info = plsc.get_sparse_core_info()
NC = _info.num_cores
NS = _info.num_subcores
NW = NC * NS
EP = E // NW
CH = 80
NCHUNK = EP // CH
NP = 10240
N_TILE = NP // NS
PAD = 64

_mesh = plsc.VectorSubcoreMesh(core_axis_name="c", subcore_axis_name="s")

ROW_BLK = 1000


def _mm_kernel(x_ref, w_ref, o_ref):
    o_ref[...] = jnp.dot(x_ref[...], w_ref[...],
                         preferred_element_type=jnp.float32)


def _pallas_matmul(x, w):
    n, h = x.shape
    m = w.shape[1]
    return pl.pallas_call(
        _mm_kernel,
        grid=(n // ROW_BLK,),
        in_specs=[
            pl.BlockSpec((ROW_BLK, h), lambda i: (i, 0)),
            pl.BlockSpec((h, m), lambda i: (0, 0)),
        ],
        out_specs=pl.BlockSpec((ROW_BLK, m), lambda i: (i, 0)),
        out_shape=jax.ShapeDtypeStruct((n, m), jnp.float32),
    )(x, w)


@functools.partial(
    pl.kernel,
    out_type=jax.ShapeDtypeStruct((NC, N), jnp.float32),
    mesh=_mesh,
    scratch_types=[
        pltpu.VMEM((CH,), jnp.int32),
        pltpu.VMEM((CH,), jnp.float32),
        pltpu.VMEM_SHARED((N,), jnp.float32),
    ],
)
def _deg_kernel(dst_hbm, zeros_hbm, out_hbm, idx_v, ones_v, acc_sh):
    cid = lax.axis_index("c")
    sid = lax.axis_index("s")
    wid = sid * NC + cid

    def initones(i, _):
        ones_v[pl.ds(i * 16, 16)] = jnp.full((16,), 1.0, jnp.float32)
        return 0
    lax.fori_loop(0, CH // 16, initones, 0)

    @pl.when(sid == 0)
    def _():
        pltpu.sync_copy(zeros_hbm, acc_sh)
    plsc.subcore_barrier()

    base = wid * EP

    def chunk(i, _):
        pltpu.sync_copy(dst_hbm.at[pl.ds(base + i * CH, CH)], idx_v)
        pltpu.sync_copy(ones_v, acc_sh.at[idx_v], add=True)
        return 0
    lax.fori_loop(0, NCHUNK, chunk, 0)
    plsc.subcore_barrier()

    @pl.when(sid == 0)
    def _():
        pltpu.sync_copy(acc_sh, out_hbm.at[cid])


@functools.partial(
    pl.kernel,
    out_type=jax.ShapeDtypeStruct((NC, NP, H), jnp.float32),
    mesh=_mesh,
    scratch_types=[
        pltpu.VMEM((CH,), jnp.int32),
        pltpu.VMEM((CH,), jnp.int32),
        pltpu.VMEM((CH, H), jnp.float32),
        pltpu.VMEM_SHARED((NP, H), jnp.float32),
        pltpu.SemaphoreType.DMA,
    ],
)
def _agg_kernel(src_hbm, dst_hbm, rows_hbm, zeros_hbm, out_hbm,
                sidx, didx, rows_v, acc_sh, sem):
    cid = lax.axis_index("c")
    sid = lax.axis_index("s")
    wid = sid * NC + cid

    pltpu.sync_copy(zeros_hbm.at[pl.ds(sid * N_TILE, N_TILE)],
                    acc_sh.at[pl.ds(sid * N_TILE, N_TILE)])
    plsc.subcore_barrier()

    base = wid * EP

    def chunk(i, _):
        off = base + i * CH
        pltpu.sync_copy(src_hbm.at[pl.ds(off, CH)], sidx)
        pltpu.sync_copy(dst_hbm.at[pl.ds(off, CH)], didx)
        pltpu.async_copy(rows_hbm.at[sidx], rows_v, sem).wait()
        pltpu.sync_copy(rows_v, acc_sh.at[didx], add=True)
        return 0
    lax.fori_loop(0, NCHUNK, chunk, 0)
    plsc.subcore_barrier()

    pltpu.sync_copy(acc_sh.at[pl.ds(sid * N_TILE, N_TILE)],
                    out_hbm.at[cid, pl.ds(sid * N_TILE, N_TILE)])


def _sc_degree(dst):
    parts = _deg_kernel(dst, jnp.zeros((N,), jnp.float32))
    return parts[0] + parts[1]


def _sc_aggregate(src, dst, rows):
    parts = _agg_kernel(src, dst, rows, jnp.zeros((NP, H), jnp.float32))
    return parts[0][:N] + parts[1][:N]


def _pad_rows(rows):
    return jnp.concatenate([rows, jnp.zeros((PAD, H), jnp.float32)], axis=0)


def _per_node_topk_mask(dst, scores, k):
    order = jnp.lexsort((-scores, dst))
    dst_s = dst[order]
    start = jnp.searchsorted(dst_s, dst_s, side='left')
    rank = jnp.arange(dst.shape[0]) - start
    mask_s = rank < k
    return jnp.zeros((dst.shape[0],), bool).at[order].set(mask_s)


def kernel(inputs, A, emb, W1, b1, W2, b2, ggnn_w, w_ih, w_hh, b_ih, b_hh):
    hidden = emb
    src, dst = A[0], A[1]

    deg = 1.0 + _sc_degree(dst)
    dinv = 1.0 / jnp.sqrt(deg)
    dcol = dinv[:, None]

    xw1 = _pallas_matmul(hidden, W1)
    agg1 = _sc_aggregate(src, dst, _pad_rows(dcol * xw1))
    h = jax.nn.relu(dcol * agg1 + dcol * dcol * xw1 + b1)

    xw2 = _pallas_matmul(h, W2)
    agg2 = _sc_aggregate(src, dst, _pad_rows(dcol * xw2))
    h = jax.nn.relu(dcol * agg2 + dcol * dcol * xw2 + b2)

    nrm = jnp.maximum(jnp.linalg.norm(h, axis=1), 1e-8)
    scores = jnp.sum(h[src] * h[dst], axis=1) / (nrm[src] * nrm[dst])
    mask = _per_node_topk_mask(dst, scores, K_TOP)

    src_m = jnp.where(mask, src, N + (jnp.arange(E, dtype=jnp.int32) & (PAD - 1)))

    x = hidden
    for i in range(L_GGNN):
        m = _pallas_matmul(x, ggnn_w[i])
        agg = _sc_aggregate(src_m, dst, _pad_rows(m))
        gi = _pallas_matmul(agg, w_ih.T) + b_ih
        gh = _pallas_matmul(x, w_hh.T) + b_hh
        i_r, i_z, i_n = jnp.split(gi, 3, axis=1)
        h_r, h_z, h_n = jnp.split(gh, 3, axis=1)
        r = jax.nn.sigmoid(i_r + h_r)
        z = jax.nn.sigmoid(i_z + h_z)
        nn_ = jnp.tanh(i_n + r * h_n)
        x = (1.0 - z) * nn_ + z * x
    return x


# speedup vs baseline: 7.8728x; 4.9496x over previous
import functools
import jax
import jax.numpy as jnp
from jax import lax
from jax.experimental import pallas as pl
from jax.experimental.pallas import tpu as pltpu
from jax.experimental.pallas import tpu_sc as plsc

N = 10000
E = 320000
H = 128
K_TOP = 15
L_GGNN = 2

NC = 2
NS = 16
NW = NC * NS
EP = E // NW
CH = 80
NCHUNK = EP // CH
NP = 10240
N_TILE = NP // NS
PAD = 64

def _mesh():
    return plsc.VectorSubcoreMesh(core_axis_name="c", subcore_axis_name="s",
                                  num_cores=NC, num_subcores=NS)

ROW_BLK = 1000


def _mm_kernel(x_ref, w_ref, o_ref):
    o_ref[...] = jnp.dot(x_ref[...], w_ref[...],
                         preferred_element_type=jnp.float32)


def _pallas_matmul(x, w):
    n, h = x.shape
    m = w.shape[1]
    return pl.pallas_call(
        _mm_kernel,
        grid=(n // ROW_BLK,),
        in_specs=[
            pl.BlockSpec((ROW_BLK, h), lambda i: (i, 0)),
            pl.BlockSpec((h, m), lambda i: (0, 0)),
        ],
        out_specs=pl.BlockSpec((ROW_BLK, m), lambda i: (i, 0)),
        out_shape=jax.ShapeDtypeStruct((n, m), jnp.float32),
    )(x, w)


def _deg_body(dst_hbm, zeros_hbm, out_hbm, idx_v, ones_v, acc_sh):
    cid = lax.axis_index("c")
    sid = lax.axis_index("s")
    wid = sid * NC + cid

    def initones(i, _):
        ones_v[pl.ds(i * 16, 16)] = jnp.full((16,), 1.0, jnp.float32)
        return 0
    lax.fori_loop(0, CH // 16, initones, 0)

    @pl.when(sid == 0)
    def _():
        pltpu.sync_copy(zeros_hbm, acc_sh)
    plsc.subcore_barrier()

    base = wid * EP

    def chunk(i, _):
        pltpu.sync_copy(dst_hbm.at[pl.ds(base + i * CH, CH)], idx_v)
        pltpu.sync_copy(ones_v, acc_sh.at[idx_v], add=True)
        return 0
    lax.fori_loop(0, NCHUNK, chunk, 0)
    plsc.subcore_barrier()

    @pl.when(sid == 0)
    def _():
        pltpu.sync_copy(acc_sh, out_hbm.at[cid])


def _agg_body(src_hbm, dst_hbm, rows_hbm, zeros_hbm, out_hbm,
                sidx, didx, rows_v, acc_sh, sem):
    cid = lax.axis_index("c")
    sid = lax.axis_index("s")
    wid = sid * NC + cid

    pltpu.sync_copy(zeros_hbm.at[pl.ds(sid * N_TILE, N_TILE)],
                    acc_sh.at[pl.ds(sid * N_TILE, N_TILE)])
    plsc.subcore_barrier()

    base = wid * EP

    def chunk(i, _):
        off = base + i * CH
        pltpu.sync_copy(src_hbm.at[pl.ds(off, CH)], sidx)
        pltpu.sync_copy(dst_hbm.at[pl.ds(off, CH)], didx)
        pltpu.async_copy(rows_hbm.at[sidx], rows_v, sem).wait()
        pltpu.sync_copy(rows_v, acc_sh.at[didx], add=True)
        return 0
    lax.fori_loop(0, NCHUNK, chunk, 0)
    plsc.subcore_barrier()

    pltpu.sync_copy(acc_sh.at[pl.ds(sid * N_TILE, N_TILE)],
                    out_hbm.at[cid, pl.ds(sid * N_TILE, N_TILE)])


@functools.lru_cache
def _deg_kernel():
    return pl.kernel(
        _deg_body,
        out_type=jax.ShapeDtypeStruct((NC, N), jnp.float32),
        mesh=_mesh(),
        compiler_params=pltpu.CompilerParams(needs_layout_passes=False),
        scratch_types=[
            pltpu.VMEM((CH,), jnp.int32),
            pltpu.VMEM((CH,), jnp.float32),
            pltpu.VMEM_SHARED((N,), jnp.float32),
        ],
    )


@functools.lru_cache
def _agg_kernel():
    return pl.kernel(
        _agg_body,
        out_type=jax.ShapeDtypeStruct((NC, NP, H), jnp.float32),
        mesh=_mesh(),
        compiler_params=pltpu.CompilerParams(needs_layout_passes=False),
        scratch_types=[
            pltpu.VMEM((CH,), jnp.int32),
            pltpu.VMEM((CH,), jnp.int32),
            pltpu.VMEM((CH, H), jnp.float32),
            pltpu.VMEM_SHARED((NP, H), jnp.float32),
            pltpu.SemaphoreType.DMA,
        ],
    )


def _sc_degree(dst):
    parts = _deg_kernel()(dst, jnp.zeros((N,), jnp.float32))
    return parts[0] + parts[1]


def _sc_aggregate(src, dst, rows):
    parts = _agg_kernel()(src, dst, rows, jnp.zeros((NP, H), jnp.float32))
    return parts[0][:N] + parts[1][:N]


def _pad_rows(rows):
    return jnp.concatenate([rows, jnp.zeros((PAD, H), jnp.float32)], axis=0)


DPO = 320
NG = DPO + 1
CCAP = 512
CELL = NW * CCAP
GPAD = CELL + 512
SENT = E
MASK_PAD = 4096
NEG = jnp.float32(-3e38)


def _place16(key16, cnt_v, start_v, sortbuf):
    lanes = lax.iota(jnp.int32, 16)
    sd, perm = plsc.sort_key_val(key16, lanes)
    sortbuf[pl.ds(1, 16)] = sd
    prev = sortbuf[pl.ds(0, 16)]
    newseg = (sd != prev) | (lanes == 0)
    segstart = plsc.cummax(jnp.where(newseg, lanes, 0))
    rank_s = lanes - segstart
    oldc = plsc.load_gather(cnt_v, [sd])
    base = plsc.load_gather(start_v, [sd])
    slot_s = base + oldc + rank_s
    sortbuf[pl.ds(17, 16)] = newseg.astype(jnp.int32)
    nxt = sortbuf[pl.ds(18, 16)]
    is_last = (lanes == 15) | (nxt == 1)
    plsc.store_scatter(cnt_v, [sd], oldc + rank_s + 1, mask=is_last)
    plsc.store_scatter(sortbuf, [perm + 34], slot_s)
    return sortbuf[pl.ds(34, 16)]


def _t1_body(src_hbm, dst_hbm, hn_hbm,
               counts_hbm, dstc_hbm, scc_hbm, srcc_hbm, idxc_hbm,
               sidx, didx, hsrc, hdst, dotbuf,
               gdst, gsc, gsrc, gidx, cnt_v, start_v, sortbuf,
               sem1, sem2):
    cid = lax.axis_index("c")
    sid = lax.axis_index("s")
    wid = sid * NC + cid
    lanes = lax.iota(jnp.int32, 16)

    for v in range(2):
        cnt_v[pl.ds(v * 16, 16)] = jnp.zeros((16,), jnp.int32)
        start_v[pl.ds(v * 16, 16)] = (lanes + v * 16) * CCAP

    def prefill(i, _):
        owner = i // (CCAP // 16)
        gdst[pl.ds(i * 16, 16)] = jnp.full((16,), owner * DPO + DPO, jnp.int32)
        gsc[pl.ds(i * 16, 16)] = jnp.full((16,), NEG, jnp.float32)
        gsrc[pl.ds(i * 16, 16)] = jnp.zeros((16,), jnp.int32)
        gidx[pl.ds(i * 16, 16)] = jnp.full((16,), E, jnp.int32) + \
            ((i * 16) % 2048) + lanes
        return 0
    lax.fori_loop(0, CELL // 16, prefill, 0)

    base = wid * EP

    def chunk(i, _):
        off = base + i * CH
        pltpu.sync_copy(src_hbm.at[pl.ds(off, CH)], sidx)
        pltpu.sync_copy(dst_hbm.at[pl.ds(off, CH)], didx)
        c1 = pltpu.async_copy(hn_hbm.at[sidx], hsrc, sem1)
        c2 = pltpu.async_copy(hn_hbm.at[didx], hdst, sem2)
        c1.wait()
        c2.wait()

        for t in range(CH // 16):
            def edot(u, _):
                e = t * 16 + u
                acc = jnp.zeros((16,), jnp.float32)
                for j in range(H // 16):
                    acc = acc + hsrc[e, pl.ds(j * 16, 16)] * \
                        hdst[e, pl.ds(j * 16, 16)]
                dotbuf[pl.ds(u * 16, 16)] = acc
                return 0
            lax.fori_loop(0, 16, edot, 0)
            sc16 = jnp.zeros((16,), jnp.float32)
            for j in range(16):
                sc16 = sc16 + plsc.load_gather(dotbuf, [lanes * 16 + j])
            d16 = didx[pl.ds(t * 16, 16)]
            s16 = sidx[pl.ds(t * 16, 16)]
            i16 = jnp.full((16,), off + t * 16, jnp.int32) + lanes
            owner = d16 // DPO
            slot = _place16(owner, cnt_v, start_v, sortbuf)
            plsc.store_scatter(gdst, [slot], d16)
            plsc.store_scatter(gsc, [slot], sc16)
            plsc.store_scatter(gsrc, [slot], s16)
            plsc.store_scatter(gidx, [slot], i16)
        return 0
    lax.fori_loop(0, NCHUNK, chunk, 0)

    pltpu.sync_copy(cnt_v, counts_hbm.at[pl.ds(wid * NW, NW)])
    pltpu.sync_copy(gdst, dstc_hbm.at[pl.ds(wid * CELL, CELL)])
    pltpu.sync_copy(gsc, scc_hbm.at[pl.ds(wid * CELL, CELL)])
    pltpu.sync_copy(gsrc, srcc_hbm.at[pl.ds(wid * CELL, CELL)])
    pltpu.sync_copy(gidx, idxc_hbm.at[pl.ds(wid * CELL, CELL)])


def _t2_body(counts_hbm, dstc_hbm, scc_hbm, srcc_hbm, idxc_hbm, deg_hbm,
               mask_hbm,
               cnts_v, degbuf, cdst, csc, csrc, cidx,
               gsc, gsrc, gidx, maskbuf, cnt_v, start_v, sortbuf,
               istage, mstage):
    cid = lax.axis_index("c")
    sid = lax.axis_index("s")
    o = sid * NC + cid
    lanes = lax.iota(jnp.int32, 16)

    pltpu.sync_copy(counts_hbm, cnts_v.at[pl.ds(0, NW * NW)])
    pltpu.sync_copy(deg_hbm.at[pl.ds(o * DPO, DPO)], degbuf.at[pl.ds(0, DPO)])

    def scan(j, c):
        v = degbuf[pl.ds(j * 16, 16)]
        incl = plsc.cumsum(v)
        start_v[pl.ds(j * 16, 16)] = incl - v + c
        return c + jnp.sum(v)
    total = lax.fori_loop(0, DPO // 16, scan, jnp.int32(0))
    start_v[pl.ds(320, 16)] = jnp.full((16,), total, jnp.int32)
    for j in range(NG // 16 + 1):
        cnt_v[pl.ds(j * 16, 16)] = jnp.zeros((16,), jnp.int32)

    def prefill(i, _):
        gidx[pl.ds(i * 16, 16)] = jnp.full((16,), E, jnp.int32) + \
            ((i * 16) % 2048) + lanes
        maskbuf[pl.ds(i * 16, 16)] = jnp.zeros((16,), jnp.float32)
        return 0
    lax.fori_loop(0, GPAD // 16, prefill, 0)

    def cell(wr, _):
        cnt = cnts_v[pl.ds(wr * NW + o, 16)][0]
        r16 = (cnt + 15) & (-16)
        coff = wr * CELL + o * CCAP
        pltpu.sync_copy(dstc_hbm.at[pl.ds(coff, CCAP)], cdst)
        pltpu.sync_copy(scc_hbm.at[pl.ds(coff, CCAP)], csc)
        pltpu.sync_copy(srcc_hbm.at[pl.ds(coff, CCAP)], csrc)
        pltpu.sync_copy(idxc_hbm.at[pl.ds(coff, CCAP)], cidx)

        def q16(q, _):
            d16 = cdst[pl.ds(q * 16, 16)]
            key = jnp.minimum(jnp.maximum(d16 - o * DPO, 0), NG - 1)
            slot = _place16(key, cnt_v, start_v, sortbuf)
            plsc.store_scatter(gsc, [slot], csc[pl.ds(q * 16, 16)])
            plsc.store_scatter(gsrc, [slot], csrc[pl.ds(q * 16, 16)])
            plsc.store_scatter(gidx, [slot], cidx[pl.ds(q * 16, 16)])
            return 0
        lax.fori_loop(0, r16 // 16, q16, 0)
        return 0
    lax.fori_loop(0, NW, cell, 0)

    def group(g, _):
        s = start_v[pl.ds(g, 16)][0]
        d = degbuf[pl.ds(g, 16)][0]

        def tchunk(t, _):
            ts = s + t * 16
            sc_t = gsc[pl.ds(ts, 16)]
            ix_t = gidx[pl.ds(ts, 16)]

            def inner(j, rank):
                scj = gsc[pl.ds(s + j, 16)][0]
                ixj = gidx[pl.ds(s + j, 16)][0]
                beats = (scj > sc_t) | ((scj == sc_t) & (ixj < ix_t))
                return rank + beats.astype(jnp.int32)
            rank = lax.fori_loop(0, d, inner, jnp.zeros((16,), jnp.int32))
            maskbuf[pl.ds(ts, 16)] = (rank < K_TOP).astype(jnp.float32)
            return 0
        lax.fori_loop(0, (d + 15) // 16, tchunk, 0)
        return 0
    lax.fori_loop(0, DPO, group, 0)

    nslots = (total + cnt_v[pl.ds(NG - 1, 16)][0] + 15) & (-16)

    def out16(c, _):
        istage[...] = gidx[pl.ds(c * 16, 16)]
        mstage[...] = maskbuf[pl.ds(c * 16, 16)]
        pltpu.sync_copy(mstage, mask_hbm.at[istage])
        return 0
    lax.fori_loop(0, nslots // 16, out16, 0)


@functools.lru_cache
def _t1_kernel():
    return pl.kernel(
        _t1_body,
        out_type=[
            jax.ShapeDtypeStruct((NW * NW,), jnp.int32),
            jax.ShapeDtypeStruct((NW * CELL,), jnp.int32),
            jax.ShapeDtypeStruct((NW * CELL,), jnp.float32),
            jax.ShapeDtypeStruct((NW * CELL,), jnp.int32),
            jax.ShapeDtypeStruct((NW * CELL,), jnp.int32),
        ],
        mesh=_mesh(),
        compiler_params=pltpu.CompilerParams(needs_layout_passes=False),
        scratch_types=[
            pltpu.VMEM((CH,), jnp.int32),
            pltpu.VMEM((CH,), jnp.int32),
            pltpu.VMEM((CH, H), jnp.float32),
            pltpu.VMEM((CH, H), jnp.float32),
            pltpu.VMEM((256,), jnp.float32),
            pltpu.VMEM((CELL,), jnp.int32),
            pltpu.VMEM((CELL,), jnp.float32),
            pltpu.VMEM((CELL,), jnp.int32),
            pltpu.VMEM((CELL,), jnp.int32),
            pltpu.VMEM((32,), jnp.int32),
            pltpu.VMEM((32,), jnp.int32),
            pltpu.VMEM((64,), jnp.int32),
            pltpu.SemaphoreType.DMA,
            pltpu.SemaphoreType.DMA,
        ],
    )


@functools.lru_cache
def _t2_kernel():
    return pl.kernel(
        _t2_body,
        out_type=jax.ShapeDtypeStruct((E + MASK_PAD,), jnp.float32),
        mesh=_mesh(),
        compiler_params=pltpu.CompilerParams(needs_layout_passes=False),
        scratch_types=[
            pltpu.VMEM((NW * NW + 16,), jnp.int32),
            pltpu.VMEM((DPO + 16,), jnp.int32),
            pltpu.VMEM((CCAP,), jnp.int32),
            pltpu.VMEM((CCAP,), jnp.float32),
            pltpu.VMEM((CCAP,), jnp.int32),
            pltpu.VMEM((CCAP,), jnp.int32),
            pltpu.VMEM((GPAD,), jnp.float32),
            pltpu.VMEM((GPAD,), jnp.int32),
            pltpu.VMEM((GPAD,), jnp.int32),
            pltpu.VMEM((GPAD,), jnp.float32),
            pltpu.VMEM((336,), jnp.int32),
            pltpu.VMEM((336,), jnp.int32),
            pltpu.VMEM((64,), jnp.int32),
            pltpu.VMEM((16,), jnp.int32),
            pltpu.VMEM((16,), jnp.float32),
        ],
    )


def kernel(inputs, A, emb, W1, b1, W2, b2, ggnn_w, w_ih, w_hh, b_ih, b_hh):
    hidden = emb
    src, dst = A[0], A[1]

    indeg = _sc_degree(dst)
    deg = 1.0 + indeg
    dinv = 1.0 / jnp.sqrt(deg)
    dcol = dinv[:, None]

    xw1 = _pallas_matmul(hidden, W1)
    agg1 = _sc_aggregate(src, dst, _pad_rows(dcol * xw1))
    h = jax.nn.relu(dcol * agg1 + dcol * dcol * xw1 + b1)

    xw2 = _pallas_matmul(h, W2)
    agg2 = _sc_aggregate(src, dst, _pad_rows(dcol * xw2))
    h = jax.nn.relu(dcol * agg2 + dcol * dcol * xw2 + b2)

    nrm = jnp.maximum(jnp.linalg.norm(h, axis=1), 1e-8)
    hn = h / nrm[:, None]
    counts, dstc, scc, srcc, idxc = _t1_kernel()(src, dst, hn)
    deg_i = jnp.zeros((NP,), jnp.int32).at[:N].set(indeg.astype(jnp.int32))
    maskc = _t2_kernel()(counts, dstc, scc, srcc, idxc, deg_i)
    mask = maskc[:E] > 0.5

    src_m = jnp.where(mask, src, N + (jnp.arange(E, dtype=jnp.int32) & (PAD - 1)))

    x = hidden
    for i in range(L_GGNN):
        m = _pallas_matmul(x, ggnn_w[i])
        agg = _sc_aggregate(src_m, dst, _pad_rows(m))
        gi = _pallas_matmul(agg, w_ih.T) + b_ih
        gh = _pallas_matmul(x, w_hh.T) + b_hh
        i_r, i_z, i_n = jnp.split(gi, 3, axis=1)
        h_r, h_z, h_n = jnp.split(gh, 3, axis=1)
        r = jax.nn.sigmoid(i_r + h_r)
        z = jax.nn.sigmoid(i_z + h_z)
        nn_ = jnp.tanh(i_n + r * h_n)
        x = (1.0 - z) * nn_ + z * x
    return x


# speedup vs baseline: 8.8589x; 1.1253x over previous
import functools
import jax
import jax.numpy as jnp
from jax import lax
from jax.experimental import pallas as pl
from jax.experimental.pallas import tpu as pltpu
from jax.experimental.pallas import tpu_sc as plsc

N = 10000
E = 320000
H = 128
K_TOP = 15
L_GGNN = 2

NC = 2
NS = 16
NW = NC * NS
EP = E // NW
CH = 80
NCHUNK = EP // CH
NP = 10240
N_TILE = NP // NS
PAD = 64

def _mesh():
    return plsc.VectorSubcoreMesh(core_axis_name="c", subcore_axis_name="s",
                                  num_cores=NC, num_subcores=NS)

ROW_BLK = 1000


def _mm_kernel(x_ref, w_ref, o_ref):
    o_ref[...] = jnp.dot(x_ref[...], w_ref[...],
                         preferred_element_type=jnp.float32)


def _pallas_matmul(x, w):
    n, h = x.shape
    m = w.shape[1]
    return pl.pallas_call(
        _mm_kernel,
        grid=(n // ROW_BLK,),
        in_specs=[
            pl.BlockSpec((ROW_BLK, h), lambda i: (i, 0)),
            pl.BlockSpec((h, m), lambda i: (0, 0)),
        ],
        out_specs=pl.BlockSpec((ROW_BLK, m), lambda i: (i, 0)),
        out_shape=jax.ShapeDtypeStruct((n, m), jnp.float32),
    )(x, w)


def _deg_body(dst_hbm, zeros_hbm, out_hbm, idx_v, ones_v, acc_sh):
    cid = lax.axis_index("c")
    sid = lax.axis_index("s")
    wid = sid * NC + cid

    def initones(i, _):
        ones_v[pl.ds(i * 16, 16)] = jnp.full((16,), 1.0, jnp.float32)
        return 0
    lax.fori_loop(0, CH // 16, initones, 0)

    @pl.when(sid == 0)
    def _():
        pltpu.sync_copy(zeros_hbm, acc_sh)
    plsc.subcore_barrier()

    base = wid * EP

    def chunk(i, _):
        pltpu.sync_copy(dst_hbm.at[pl.ds(base + i * CH, CH)], idx_v)
        pltpu.sync_copy(ones_v, acc_sh.at[idx_v], add=True)
        return 0
    lax.fori_loop(0, NCHUNK, chunk, 0)
    plsc.subcore_barrier()

    @pl.when(sid == 0)
    def _():
        pltpu.sync_copy(acc_sh, out_hbm.at[cid])


def _agg_body(src_hbm, dst_hbm, rows_hbm, zeros_hbm, out_hbm,
                sidx0, didx0, rows0, sidx1, didx1, rows1, acc_sh,
                sem0, sem1):
    cid = lax.axis_index("c")
    sid = lax.axis_index("s")
    wid = sid * NC + cid

    pltpu.sync_copy(zeros_hbm.at[pl.ds(sid * N_TILE, N_TILE)],
                    acc_sh.at[pl.ds(sid * N_TILE, N_TILE)])
    plsc.subcore_barrier()

    base = wid * EP

    def chunk2(i, _):
        off0 = base + (2 * i) * CH
        off1 = off0 + CH
        pltpu.sync_copy(src_hbm.at[pl.ds(off0, CH)], sidx0)
        pltpu.sync_copy(dst_hbm.at[pl.ds(off0, CH)], didx0)
        c0 = pltpu.async_copy(rows_hbm.at[sidx0], rows0, sem0)
        pltpu.sync_copy(src_hbm.at[pl.ds(off1, CH)], sidx1)
        pltpu.sync_copy(dst_hbm.at[pl.ds(off1, CH)], didx1)
        c1 = pltpu.async_copy(rows_hbm.at[sidx1], rows1, sem1)
        c0.wait()
        pltpu.sync_copy(rows0, acc_sh.at[didx0], add=True)
        c1.wait()
        pltpu.sync_copy(rows1, acc_sh.at[didx1], add=True)
        return 0
    lax.fori_loop(0, NCHUNK // 2, chunk2, 0)
    off = base + (NCHUNK - 1) * CH
    pltpu.sync_copy(src_hbm.at[pl.ds(off, CH)], sidx0)
    pltpu.sync_copy(dst_hbm.at[pl.ds(off, CH)], didx0)
    pltpu.async_copy(rows_hbm.at[sidx0], rows0, sem0).wait()
    pltpu.sync_copy(rows0, acc_sh.at[didx0], add=True)
    plsc.subcore_barrier()

    pltpu.sync_copy(acc_sh.at[pl.ds(sid * N_TILE, N_TILE)],
                    out_hbm.at[cid, pl.ds(sid * N_TILE, N_TILE)])


@functools.lru_cache
def _deg_kernel():
    return pl.kernel(
        _deg_body,
        out_type=jax.ShapeDtypeStruct((NC, N), jnp.float32),
        mesh=_mesh(),
        compiler_params=pltpu.CompilerParams(needs_layout_passes=False),
        scratch_types=[
            pltpu.VMEM((CH,), jnp.int32),
            pltpu.VMEM((CH,), jnp.float32),
            pltpu.VMEM_SHARED((N,), jnp.float32),
        ],
    )


@functools.lru_cache
def _agg_kernel():
    return pl.kernel(
        _agg_body,
        out_type=jax.ShapeDtypeStruct((NC, NP, H), jnp.float32),
        mesh=_mesh(),
        compiler_params=pltpu.CompilerParams(needs_layout_passes=False),
        scratch_types=[
            pltpu.VMEM((CH,), jnp.int32),
            pltpu.VMEM((CH,), jnp.int32),
            pltpu.VMEM((CH, H), jnp.float32),
            pltpu.VMEM((CH,), jnp.int32),
            pltpu.VMEM((CH,), jnp.int32),
            pltpu.VMEM((CH, H), jnp.float32),
            pltpu.VMEM_SHARED((NP, H), jnp.float32),
            pltpu.SemaphoreType.DMA,
            pltpu.SemaphoreType.DMA,
        ],
    )


def _sc_degree(dst):
    parts = _deg_kernel()(dst, jnp.zeros((N,), jnp.float32))
    return parts[0] + parts[1]


def _sc_aggregate(src, dst, rows):
    parts = _agg_kernel()(src, dst, rows, jnp.zeros((NP, H), jnp.float32))
    return parts[0][:N] + parts[1][:N]


def _pad_rows(rows):
    return jnp.concatenate([rows, jnp.zeros((PAD, H), jnp.float32)], axis=0)


DPO = 320
NG = DPO + 1
CCAP = 512
CELL = NW * CCAP
GPAD = CELL + 512
SENT = E
MASK_PAD = 4096
NEG = jnp.float32(-3e38)


def _place16(key16, cnt_v, start_v, sortbuf):
    lanes = lax.iota(jnp.int32, 16)
    sd, perm = plsc.sort_key_val(key16, lanes)
    sortbuf[pl.ds(1, 16)] = sd
    prev = sortbuf[pl.ds(0, 16)]
    newseg = (sd != prev) | (lanes == 0)
    segstart = plsc.cummax(jnp.where(newseg, lanes, 0))
    rank_s = lanes - segstart
    oldc = plsc.load_gather(cnt_v, [sd])
    base = plsc.load_gather(start_v, [sd])
    slot_s = base + oldc + rank_s
    sortbuf[pl.ds(17, 16)] = newseg.astype(jnp.int32)
    nxt = sortbuf[pl.ds(18, 16)]
    is_last = (lanes == 15) | (nxt == 1)
    plsc.store_scatter(cnt_v, [sd], oldc + rank_s + 1, mask=is_last)
    plsc.store_scatter(sortbuf, [perm + 34], slot_s)
    return sortbuf[pl.ds(34, 16)]


def _t1_body(src_hbm, dst_hbm, hn_hbm,
               counts_hbm, dstc_hbm, scc_hbm, srcc_hbm, idxc_hbm,
               sidx, didx, hsrc, hdst, dotbuf,
               gdst, gsc, gsrc, gidx, cnt_v, start_v, sortbuf,
               sem1, sem2):
    cid = lax.axis_index("c")
    sid = lax.axis_index("s")
    wid = sid * NC + cid
    lanes = lax.iota(jnp.int32, 16)

    for v in range(2):
        cnt_v[pl.ds(v * 16, 16)] = jnp.zeros((16,), jnp.int32)
        start_v[pl.ds(v * 16, 16)] = (lanes + v * 16) * CCAP

    def prefill(i, _):
        owner = i // (CCAP // 16)
        gdst[pl.ds(i * 16, 16)] = jnp.full((16,), owner * DPO + DPO, jnp.int32)
        gsc[pl.ds(i * 16, 16)] = jnp.full((16,), NEG, jnp.float32)
        gsrc[pl.ds(i * 16, 16)] = jnp.zeros((16,), jnp.int32)
        gidx[pl.ds(i * 16, 16)] = jnp.full((16,), E, jnp.int32) + \
            ((i * 16) % 2048) + lanes
        return 0
    lax.fori_loop(0, CELL // 16, prefill, 0)

    base = wid * EP

    def chunk(i, _):
        off = base + i * CH
        pltpu.sync_copy(src_hbm.at[pl.ds(off, CH)], sidx)
        pltpu.sync_copy(dst_hbm.at[pl.ds(off, CH)], didx)
        c1 = pltpu.async_copy(hn_hbm.at[sidx], hsrc, sem1)
        c2 = pltpu.async_copy(hn_hbm.at[didx], hdst, sem2)
        c1.wait()
        c2.wait()

        for t in range(CH // 16):
            def edot(u, _):
                e = t * 16 + u
                acc = jnp.zeros((16,), jnp.float32)
                for j in range(H // 16):
                    acc = acc + hsrc[e, pl.ds(j * 16, 16)] * \
                        hdst[e, pl.ds(j * 16, 16)]
                dotbuf[pl.ds(u * 16, 16)] = acc
                return 0
            lax.fori_loop(0, 16, edot, 0)
            sc16 = jnp.zeros((16,), jnp.float32)
            for j in range(16):
                sc16 = sc16 + plsc.load_gather(dotbuf, [lanes * 16 + j])
            d16 = didx[pl.ds(t * 16, 16)]
            s16 = sidx[pl.ds(t * 16, 16)]
            i16 = jnp.full((16,), off + t * 16, jnp.int32) + lanes
            owner = d16 // DPO
            slot = _place16(owner, cnt_v, start_v, sortbuf)
            plsc.store_scatter(gdst, [slot], d16)
            plsc.store_scatter(gsc, [slot], sc16)
            plsc.store_scatter(gsrc, [slot], s16)
            plsc.store_scatter(gidx, [slot], i16)
        return 0
    lax.fori_loop(0, NCHUNK, chunk, 0)

    pltpu.sync_copy(cnt_v, counts_hbm.at[pl.ds(wid * NW, NW)])
    pltpu.sync_copy(gdst, dstc_hbm.at[pl.ds(wid * CELL, CELL)])
    pltpu.sync_copy(gsc, scc_hbm.at[pl.ds(wid * CELL, CELL)])
    pltpu.sync_copy(gsrc, srcc_hbm.at[pl.ds(wid * CELL, CELL)])
    pltpu.sync_copy(gidx, idxc_hbm.at[pl.ds(wid * CELL, CELL)])


def _t2_body(counts_hbm, dstc_hbm, scc_hbm, srcc_hbm, idxc_hbm, deg_hbm,
               mask_hbm,
               cnts_v, degbuf, cdst, csc, csrc, cidx,
               gsc, gsrc, gidx, maskbuf, cnt_v, start_v, sortbuf,
               istage, mstage):
    cid = lax.axis_index("c")
    sid = lax.axis_index("s")
    o = sid * NC + cid
    lanes = lax.iota(jnp.int32, 16)

    pltpu.sync_copy(counts_hbm, cnts_v.at[pl.ds(0, NW * NW)])
    pltpu.sync_copy(deg_hbm.at[pl.ds(o * DPO, DPO)], degbuf.at[pl.ds(0, DPO)])

    def scan(j, c):
        v = degbuf[pl.ds(j * 16, 16)]
        incl = plsc.cumsum(v)
        start_v[pl.ds(j * 16, 16)] = incl - v + c
        return c + jnp.sum(v)
    total = lax.fori_loop(0, DPO // 16, scan, jnp.int32(0))
    start_v[pl.ds(320, 16)] = jnp.full((16,), total, jnp.int32)
    for j in range(NG // 16 + 1):
        cnt_v[pl.ds(j * 16, 16)] = jnp.zeros((16,), jnp.int32)

    def prefill(i, _):
        gidx[pl.ds(i * 16, 16)] = jnp.full((16,), E, jnp.int32) + \
            ((i * 16) % 2048) + lanes
        maskbuf[pl.ds(i * 16, 16)] = jnp.zeros((16,), jnp.float32)
        return 0
    lax.fori_loop(0, GPAD // 16, prefill, 0)

    def cell(wr, _):
        cnt = cnts_v[pl.ds(wr * NW + o, 16)][0]
        r16 = (cnt + 15) & (-16)
        coff = wr * CELL + o * CCAP
        pltpu.sync_copy(dstc_hbm.at[pl.ds(coff, CCAP)], cdst)
        pltpu.sync_copy(scc_hbm.at[pl.ds(coff, CCAP)], csc)
        pltpu.sync_copy(srcc_hbm.at[pl.ds(coff, CCAP)], csrc)
        pltpu.sync_copy(idxc_hbm.at[pl.ds(coff, CCAP)], cidx)

        def q16(q, _):
            d16 = cdst[pl.ds(q * 16, 16)]
            key = jnp.minimum(jnp.maximum(d16 - o * DPO, 0), NG - 1)
            slot = _place16(key, cnt_v, start_v, sortbuf)
            plsc.store_scatter(gsc, [slot], csc[pl.ds(q * 16, 16)])
            plsc.store_scatter(gsrc, [slot], csrc[pl.ds(q * 16, 16)])
            plsc.store_scatter(gidx, [slot], cidx[pl.ds(q * 16, 16)])
            return 0
        lax.fori_loop(0, r16 // 16, q16, 0)
        return 0
    lax.fori_loop(0, NW, cell, 0)

    def group(g, _):
        s = start_v[pl.ds(g, 16)][0]
        d = degbuf[pl.ds(g, 16)][0]

        def tchunk(t, _):
            ts = s + t * 16
            sc_t = gsc[pl.ds(ts, 16)]
            ix_t = gidx[pl.ds(ts, 16)]

            def inner(j, rank):
                scj = gsc[pl.ds(s + j, 16)][0]
                ixj = gidx[pl.ds(s + j, 16)][0]
                beats = (scj > sc_t) | ((scj == sc_t) & (ixj < ix_t))
                return rank + beats.astype(jnp.int32)
            rank = lax.fori_loop(0, d, inner, jnp.zeros((16,), jnp.int32))
            maskbuf[pl.ds(ts, 16)] = (rank < K_TOP).astype(jnp.float32)
            return 0
        lax.fori_loop(0, (d + 15) // 16, tchunk, 0)
        return 0
    lax.fori_loop(0, DPO, group, 0)

    nslots = (total + cnt_v[pl.ds(NG - 1, 16)][0] + 15) & (-16)

    def out16(c, _):
        istage[...] = gidx[pl.ds(c * 16, 16)]
        mstage[...] = maskbuf[pl.ds(c * 16, 16)]
        pltpu.sync_copy(mstage, mask_hbm.at[istage])
        return 0
    lax.fori_loop(0, nslots // 16, out16, 0)


@functools.lru_cache
def _t1_kernel():
    return pl.kernel(
        _t1_body,
        out_type=[
            jax.ShapeDtypeStruct((NW * NW,), jnp.int32),
            jax.ShapeDtypeStruct((NW * CELL,), jnp.int32),
            jax.ShapeDtypeStruct((NW * CELL,), jnp.float32),
            jax.ShapeDtypeStruct((NW * CELL,), jnp.int32),
            jax.ShapeDtypeStruct((NW * CELL,), jnp.int32),
        ],
        mesh=_mesh(),
        compiler_params=pltpu.CompilerParams(needs_layout_passes=False),
        scratch_types=[
            pltpu.VMEM((CH,), jnp.int32),
            pltpu.VMEM((CH,), jnp.int32),
            pltpu.VMEM((CH, H), jnp.float32),
            pltpu.VMEM((CH, H), jnp.float32),
            pltpu.VMEM((256,), jnp.float32),
            pltpu.VMEM((CELL,), jnp.int32),
            pltpu.VMEM((CELL,), jnp.float32),
            pltpu.VMEM((CELL,), jnp.int32),
            pltpu.VMEM((CELL,), jnp.int32),
            pltpu.VMEM((32,), jnp.int32),
            pltpu.VMEM((32,), jnp.int32),
            pltpu.VMEM((64,), jnp.int32),
            pltpu.SemaphoreType.DMA,
            pltpu.SemaphoreType.DMA,
        ],
    )


@functools.lru_cache
def _t2_kernel():
    return pl.kernel(
        _t2_body,
        out_type=jax.ShapeDtypeStruct((E + MASK_PAD,), jnp.float32),
        mesh=_mesh(),
        compiler_params=pltpu.CompilerParams(needs_layout_passes=False),
        scratch_types=[
            pltpu.VMEM((NW * NW + 16,), jnp.int32),
            pltpu.VMEM((DPO + 16,), jnp.int32),
            pltpu.VMEM((CCAP,), jnp.int32),
            pltpu.VMEM((CCAP,), jnp.float32),
            pltpu.VMEM((CCAP,), jnp.int32),
            pltpu.VMEM((CCAP,), jnp.int32),
            pltpu.VMEM((GPAD,), jnp.float32),
            pltpu.VMEM((GPAD,), jnp.int32),
            pltpu.VMEM((GPAD,), jnp.int32),
            pltpu.VMEM((GPAD,), jnp.float32),
            pltpu.VMEM((336,), jnp.int32),
            pltpu.VMEM((336,), jnp.int32),
            pltpu.VMEM((64,), jnp.int32),
            pltpu.VMEM((16,), jnp.int32),
            pltpu.VMEM((16,), jnp.float32),
        ],
    )


def kernel(inputs, A, emb, W1, b1, W2, b2, ggnn_w, w_ih, w_hh, b_ih, b_hh):
    hidden = emb
    src, dst = A[0], A[1]

    indeg = _sc_degree(dst)
    deg = 1.0 + indeg
    dinv = 1.0 / jnp.sqrt(deg)
    dcol = dinv[:, None]

    xw1 = _pallas_matmul(hidden, W1)
    agg1 = _sc_aggregate(src, dst, _pad_rows(dcol * xw1))
    h = jax.nn.relu(dcol * agg1 + dcol * dcol * xw1 + b1)

    xw2 = _pallas_matmul(h, W2)
    agg2 = _sc_aggregate(src, dst, _pad_rows(dcol * xw2))
    h = jax.nn.relu(dcol * agg2 + dcol * dcol * xw2 + b2)

    nrm = jnp.maximum(jnp.linalg.norm(h, axis=1), 1e-8)
    hn = h / nrm[:, None]
    counts, dstc, scc, srcc, idxc = _t1_kernel()(src, dst, hn)
    deg_i = jnp.zeros((NP,), jnp.int32).at[:N].set(indeg.astype(jnp.int32))
    maskc = _t2_kernel()(counts, dstc, scc, srcc, idxc, deg_i)
    mask = maskc[:E] > 0.5

    src_m = jnp.where(mask, src, N + (jnp.arange(E, dtype=jnp.int32) & (PAD - 1)))

    x = hidden
    for i in range(L_GGNN):
        m = _pallas_matmul(x, ggnn_w[i])
        agg = _sc_aggregate(src_m, dst, _pad_rows(m))
        gi = _pallas_matmul(agg, w_ih.T) + b_ih
        gh = _pallas_matmul(x, w_hh.T) + b_hh
        i_r, i_z, i_n = jnp.split(gi, 3, axis=1)
        h_r, h_z, h_n = jnp.split(gh, 3, axis=1)
        r = jax.nn.sigmoid(i_r + h_r)
        z = jax.nn.sigmoid(i_z + h_z)
        nn_ = jnp.tanh(i_n + r * h_n)
        x = (1.0 - z) * nn_ + z * x
    return x


# speedup vs baseline: 9.2190x; 1.0407x over previous
import functools
import jax
import jax.numpy as jnp
from jax import lax
from jax.experimental import pallas as pl
from jax.experimental.pallas import tpu as pltpu
from jax.experimental.pallas import tpu_sc as plsc

N = 10000
E = 320000
H = 128
K_TOP = 15
L_GGNN = 2

NC = 2
NS = 16
NW = NC * NS
EP = E // NW
CH = 80
NCHUNK = EP // CH
NP = 10240
N_TILE = NP // NS
PAD = 64

def _mesh():
    return plsc.VectorSubcoreMesh(core_axis_name="c", subcore_axis_name="s",
                                  num_cores=NC, num_subcores=NS)

ROW_BLK = 1000


def _mm_kernel(x_ref, w_ref, o_ref):
    o_ref[...] = jnp.dot(x_ref[...], w_ref[...],
                         preferred_element_type=jnp.float32)


def _pallas_matmul(x, w):
    n, h = x.shape
    m = w.shape[1]
    return pl.pallas_call(
        _mm_kernel,
        grid=(n // ROW_BLK,),
        in_specs=[
            pl.BlockSpec((ROW_BLK, h), lambda i: (i, 0)),
            pl.BlockSpec((h, m), lambda i: (0, 0)),
        ],
        out_specs=pl.BlockSpec((ROW_BLK, m), lambda i: (i, 0)),
        out_shape=jax.ShapeDtypeStruct((n, m), jnp.float32),
    )(x, w)


def _deg_body(dst_hbm, zeros_hbm, out_hbm, idx_v, ones_v, acc_sh):
    cid = lax.axis_index("c")
    sid = lax.axis_index("s")
    wid = sid * NC + cid

    def initones(i, _):
        ones_v[pl.ds(i * 16, 16)] = jnp.full((16,), 1.0, jnp.float32)
        return 0
    lax.fori_loop(0, CH // 16, initones, 0)

    @pl.when(sid == 0)
    def _():
        pltpu.sync_copy(zeros_hbm, acc_sh)
    plsc.subcore_barrier()

    base = wid * EP

    def chunk(i, _):
        pltpu.sync_copy(dst_hbm.at[pl.ds(base + i * CH, CH)], idx_v)
        pltpu.sync_copy(ones_v, acc_sh.at[idx_v], add=True)
        return 0
    lax.fori_loop(0, NCHUNK, chunk, 0)
    plsc.subcore_barrier()

    @pl.when(sid == 0)
    def _():
        pltpu.sync_copy(acc_sh, out_hbm.at[cid])


def _agg_body(src_hbm, dst_hbm, rows_hbm, zeros_hbm, out_hbm,
                sidx0, didx0, rows0, sidx1, didx1, rows1, acc_sh,
                sem0, sem1):
    cid = lax.axis_index("c")
    sid = lax.axis_index("s")
    wid = sid * NC + cid

    pltpu.sync_copy(zeros_hbm.at[pl.ds(sid * N_TILE, N_TILE)],
                    acc_sh.at[pl.ds(sid * N_TILE, N_TILE)])
    plsc.subcore_barrier()

    base = wid * EP

    def chunk2(i, _):
        off0 = base + (2 * i) * CH
        off1 = off0 + CH
        pltpu.sync_copy(src_hbm.at[pl.ds(off0, CH)], sidx0)
        pltpu.sync_copy(dst_hbm.at[pl.ds(off0, CH)], didx0)
        c0 = pltpu.async_copy(rows_hbm.at[sidx0], rows0, sem0)
        pltpu.sync_copy(src_hbm.at[pl.ds(off1, CH)], sidx1)
        pltpu.sync_copy(dst_hbm.at[pl.ds(off1, CH)], didx1)
        c1 = pltpu.async_copy(rows_hbm.at[sidx1], rows1, sem1)
        c0.wait()
        pltpu.sync_copy(rows0, acc_sh.at[didx0], add=True)
        c1.wait()
        pltpu.sync_copy(rows1, acc_sh.at[didx1], add=True)
        return 0
    lax.fori_loop(0, NCHUNK // 2, chunk2, 0)
    off = base + (NCHUNK - 1) * CH
    pltpu.sync_copy(src_hbm.at[pl.ds(off, CH)], sidx0)
    pltpu.sync_copy(dst_hbm.at[pl.ds(off, CH)], didx0)
    pltpu.async_copy(rows_hbm.at[sidx0], rows0, sem0).wait()
    pltpu.sync_copy(rows0, acc_sh.at[didx0], add=True)
    plsc.subcore_barrier()

    pltpu.sync_copy(acc_sh.at[pl.ds(sid * N_TILE, N_TILE)],
                    out_hbm.at[cid, pl.ds(sid * N_TILE, N_TILE)])


@functools.lru_cache
def _deg_kernel():
    return pl.kernel(
        _deg_body,
        out_type=jax.ShapeDtypeStruct((NC, N), jnp.float32),
        mesh=_mesh(),
        compiler_params=pltpu.CompilerParams(needs_layout_passes=False),
        scratch_types=[
            pltpu.VMEM((CH,), jnp.int32),
            pltpu.VMEM((CH,), jnp.float32),
            pltpu.VMEM_SHARED((N,), jnp.float32),
        ],
    )


@functools.lru_cache
def _agg_kernel():
    return pl.kernel(
        _agg_body,
        out_type=jax.ShapeDtypeStruct((NC, NP, H), jnp.float32),
        mesh=_mesh(),
        compiler_params=pltpu.CompilerParams(needs_layout_passes=False),
        scratch_types=[
            pltpu.VMEM((CH,), jnp.int32),
            pltpu.VMEM((CH,), jnp.int32),
            pltpu.VMEM((CH, H), jnp.float32),
            pltpu.VMEM((CH,), jnp.int32),
            pltpu.VMEM((CH,), jnp.int32),
            pltpu.VMEM((CH, H), jnp.float32),
            pltpu.VMEM_SHARED((NP, H), jnp.float32),
            pltpu.SemaphoreType.DMA,
            pltpu.SemaphoreType.DMA,
        ],
    )


def _sc_degree(dst):
    parts = _deg_kernel()(dst, jnp.zeros((N,), jnp.float32))
    return parts[0] + parts[1]


def _sc_aggregate(src, dst, rows):
    parts = _agg_kernel()(src, dst, rows, jnp.zeros((NP, H), jnp.float32))
    return parts[0][:N] + parts[1][:N]


def _pad_rows(rows):
    return jnp.concatenate([rows, jnp.zeros((PAD, H), jnp.float32)], axis=0)


DPO = 320
NG = DPO + 1
CCAP = 512
CELL = NW * CCAP
GPAD = CELL + 512
SENT = E
MASK_PAD = 4096
NEG = jnp.float32(-3e38)


def _place16(key16, cnt_v, start_v, sortbuf):
    lanes = lax.iota(jnp.int32, 16)
    sd, perm = plsc.sort_key_val(key16, lanes)
    sortbuf[pl.ds(1, 16)] = sd
    prev = sortbuf[pl.ds(0, 16)]
    newseg = (sd != prev) | (lanes == 0)
    segstart = plsc.cummax(jnp.where(newseg, lanes, 0))
    rank_s = lanes - segstart
    oldc = plsc.load_gather(cnt_v, [sd])
    base = plsc.load_gather(start_v, [sd])
    slot_s = base + oldc + rank_s
    sortbuf[pl.ds(17, 16)] = newseg.astype(jnp.int32)
    nxt = sortbuf[pl.ds(18, 16)]
    is_last = (lanes == 15) | (nxt == 1)
    plsc.store_scatter(cnt_v, [sd], oldc + rank_s + 1, mask=is_last)
    plsc.store_scatter(sortbuf, [perm + 34], slot_s)
    return sortbuf[pl.ds(34, 16)]


def _t1_body(src_hbm, dst_hbm, hn_hbm,
               counts_hbm, dstc_hbm, scc_hbm, srcc_hbm, idxc_hbm,
               sidx, didx, hsrc, hdst, sidxb, didxb, hsrcb, hdstb, dotbuf,
               gdst, gsc, gsrc, gidx, cnt_v, start_v, sortbuf,
               sem1, sem2, sem3, sem4):
    cid = lax.axis_index("c")
    sid = lax.axis_index("s")
    wid = sid * NC + cid
    lanes = lax.iota(jnp.int32, 16)

    for v in range(2):
        cnt_v[pl.ds(v * 16, 16)] = jnp.zeros((16,), jnp.int32)
        start_v[pl.ds(v * 16, 16)] = (lanes + v * 16) * CCAP

    def prefill(i, _):
        owner = i // (CCAP // 16)
        gdst[pl.ds(i * 16, 16)] = jnp.full((16,), owner * DPO + DPO, jnp.int32)
        gsc[pl.ds(i * 16, 16)] = jnp.full((16,), NEG, jnp.float32)
        gsrc[pl.ds(i * 16, 16)] = jnp.zeros((16,), jnp.int32)
        gidx[pl.ds(i * 16, 16)] = jnp.full((16,), E, jnp.int32) + \
            ((i * 16) % 2048) + lanes
        return 0
    lax.fori_loop(0, CELL // 16, prefill, 0)

    base = wid * EP

    def fire(off, sx, dx, hs, hd, sa, sb):
        pltpu.sync_copy(src_hbm.at[pl.ds(off, CH)], sx)
        pltpu.sync_copy(dst_hbm.at[pl.ds(off, CH)], dx)
        c1 = pltpu.async_copy(hn_hbm.at[sx], hs, sa)
        c2 = pltpu.async_copy(hn_hbm.at[dx], hd, sb)
        return c1, c2

    def process(off, sx, dx, hs, hd):
        for t in range(CH // 16):
            def edot(u, _):
                e = t * 16 + u
                acc = jnp.zeros((16,), jnp.float32)
                for j in range(H // 16):
                    acc = acc + hs[e, pl.ds(j * 16, 16)] * \
                        hd[e, pl.ds(j * 16, 16)]
                dotbuf[pl.ds(u * 16, 16)] = acc
                return 0
            lax.fori_loop(0, 16, edot, 0)
            sc16 = jnp.zeros((16,), jnp.float32)
            for j in range(16):
                sc16 = sc16 + plsc.load_gather(dotbuf, [lanes * 16 + j])
            d16 = dx[pl.ds(t * 16, 16)]
            s16 = sx[pl.ds(t * 16, 16)]
            i16 = jnp.full((16,), off + t * 16, jnp.int32) + lanes
            owner = d16 // DPO
            slot = _place16(owner, cnt_v, start_v, sortbuf)
            plsc.store_scatter(gdst, [slot], d16)
            plsc.store_scatter(gsc, [slot], sc16)
            plsc.store_scatter(gsrc, [slot], s16)
            plsc.store_scatter(gidx, [slot], i16)

    def chunk2(i, _):
        off0 = base + (2 * i) * CH
        off1 = off0 + CH
        a1, a2 = fire(off0, sidx, didx, hsrc, hdst, sem1, sem2)
        b1, b2 = fire(off1, sidxb, didxb, hsrcb, hdstb, sem3, sem4)
        a1.wait()
        a2.wait()
        process(off0, sidx, didx, hsrc, hdst)
        b1.wait()
        b2.wait()
        process(off1, sidxb, didxb, hsrcb, hdstb)
        return 0
    lax.fori_loop(0, NCHUNK // 2, chunk2, 0)
    offt = base + (NCHUNK - 1) * CH
    t1, t2 = fire(offt, sidx, didx, hsrc, hdst, sem1, sem2)
    t1.wait()
    t2.wait()
    process(offt, sidx, didx, hsrc, hdst)

    pltpu.sync_copy(cnt_v, counts_hbm.at[pl.ds(wid * NW, NW)])
    pltpu.sync_copy(gdst, dstc_hbm.at[pl.ds(wid * CELL, CELL)])
    pltpu.sync_copy(gsc, scc_hbm.at[pl.ds(wid * CELL, CELL)])
    pltpu.sync_copy(gsrc, srcc_hbm.at[pl.ds(wid * CELL, CELL)])
    pltpu.sync_copy(gidx, idxc_hbm.at[pl.ds(wid * CELL, CELL)])


def _t2_body(counts_hbm, dstc_hbm, scc_hbm, srcc_hbm, idxc_hbm, deg_hbm,
               mask_hbm,
               cnts_v, degbuf, cdst, csc, csrc, cidx,
               gsc, gsrc, gidx, maskbuf, cnt_v, start_v, sortbuf,
               istage, mstage):
    cid = lax.axis_index("c")
    sid = lax.axis_index("s")
    o = sid * NC + cid
    lanes = lax.iota(jnp.int32, 16)

    pltpu.sync_copy(counts_hbm, cnts_v.at[pl.ds(0, NW * NW)])
    pltpu.sync_copy(deg_hbm.at[pl.ds(o * DPO, DPO)], degbuf.at[pl.ds(0, DPO)])

    def scan(j, c):
        v = degbuf[pl.ds(j * 16, 16)]
        incl = plsc.cumsum(v)
        start_v[pl.ds(j * 16, 16)] = incl - v + c
        return c + jnp.sum(v)
    total = lax.fori_loop(0, DPO // 16, scan, jnp.int32(0))
    start_v[pl.ds(320, 16)] = jnp.full((16,), total, jnp.int32)
    for j in range(NG // 16 + 1):
        cnt_v[pl.ds(j * 16, 16)] = jnp.zeros((16,), jnp.int32)

    def prefill(i, _):
        gidx[pl.ds(i * 16, 16)] = jnp.full((16,), E, jnp.int32) + \
            ((i * 16) % 2048) + lanes
        maskbuf[pl.ds(i * 16, 16)] = jnp.zeros((16,), jnp.float32)
        return 0
    lax.fori_loop(0, GPAD // 16, prefill, 0)

    def cell(wr, _):
        cnt = cnts_v[pl.ds(wr * NW + o, 16)][0]
        r16 = (cnt + 15) & (-16)
        coff = wr * CELL + o * CCAP
        pltpu.sync_copy(dstc_hbm.at[pl.ds(coff, CCAP)], cdst)
        pltpu.sync_copy(scc_hbm.at[pl.ds(coff, CCAP)], csc)
        pltpu.sync_copy(srcc_hbm.at[pl.ds(coff, CCAP)], csrc)
        pltpu.sync_copy(idxc_hbm.at[pl.ds(coff, CCAP)], cidx)

        def q16(q, _):
            d16 = cdst[pl.ds(q * 16, 16)]
            key = jnp.minimum(jnp.maximum(d16 - o * DPO, 0), NG - 1)
            slot = _place16(key, cnt_v, start_v, sortbuf)
            plsc.store_scatter(gsc, [slot], csc[pl.ds(q * 16, 16)])
            plsc.store_scatter(gsrc, [slot], csrc[pl.ds(q * 16, 16)])
            plsc.store_scatter(gidx, [slot], cidx[pl.ds(q * 16, 16)])
            return 0
        lax.fori_loop(0, r16 // 16, q16, 0)
        return 0
    lax.fori_loop(0, NW, cell, 0)

    def group(g, _):
        s = start_v[pl.ds(g, 16)][0]
        d = degbuf[pl.ds(g, 16)][0]

        def tchunk(t, _):
            ts = s + t * 16
            sc_t = gsc[pl.ds(ts, 16)]
            ix_t = gidx[pl.ds(ts, 16)]

            def inner(j, rank):
                scj = gsc[pl.ds(s + j, 16)][0]
                ixj = gidx[pl.ds(s + j, 16)][0]
                beats = (scj > sc_t) | ((scj == sc_t) & (ixj < ix_t))
                return rank + beats.astype(jnp.int32)
            rank = lax.fori_loop(0, d, inner, jnp.zeros((16,), jnp.int32))
            maskbuf[pl.ds(ts, 16)] = (rank < K_TOP).astype(jnp.float32)
            return 0
        lax.fori_loop(0, (d + 15) // 16, tchunk, 0)
        return 0
    lax.fori_loop(0, DPO, group, 0)

    nslots = (total + cnt_v[pl.ds(NG - 1, 16)][0] + 15) & (-16)

    def out16(c, _):
        istage[...] = gidx[pl.ds(c * 16, 16)]
        mstage[...] = maskbuf[pl.ds(c * 16, 16)]
        pltpu.sync_copy(mstage, mask_hbm.at[istage])
        return 0
    lax.fori_loop(0, nslots // 16, out16, 0)


@functools.lru_cache
def _t1_kernel():
    return pl.kernel(
        _t1_body,
        out_type=[
            jax.ShapeDtypeStruct((NW * NW,), jnp.int32),
            jax.ShapeDtypeStruct((NW * CELL,), jnp.int32),
            jax.ShapeDtypeStruct((NW * CELL,), jnp.float32),
            jax.ShapeDtypeStruct((NW * CELL,), jnp.int32),
            jax.ShapeDtypeStruct((NW * CELL,), jnp.int32),
        ],
        mesh=_mesh(),
        compiler_params=pltpu.CompilerParams(needs_layout_passes=False),
        scratch_types=[
            pltpu.VMEM((CH,), jnp.int32),
            pltpu.VMEM((CH,), jnp.int32),
            pltpu.VMEM((CH, H), jnp.float32),
            pltpu.VMEM((CH, H), jnp.float32),
            pltpu.VMEM((CH,), jnp.int32),
            pltpu.VMEM((CH,), jnp.int32),
            pltpu.VMEM((CH, H), jnp.float32),
            pltpu.VMEM((CH, H), jnp.float32),
            pltpu.VMEM((256,), jnp.float32),
            pltpu.VMEM((CELL,), jnp.int32),
            pltpu.VMEM((CELL,), jnp.float32),
            pltpu.VMEM((CELL,), jnp.int32),
            pltpu.VMEM((CELL,), jnp.int32),
            pltpu.VMEM((32,), jnp.int32),
            pltpu.VMEM((32,), jnp.int32),
            pltpu.VMEM((64,), jnp.int32),
            pltpu.SemaphoreType.DMA,
            pltpu.SemaphoreType.DMA,
            pltpu.SemaphoreType.DMA,
            pltpu.SemaphoreType.DMA,
        ],
    )


@functools.lru_cache
def _t2_kernel():
    return pl.kernel(
        _t2_body,
        out_type=jax.ShapeDtypeStruct((E + MASK_PAD,), jnp.float32),
        mesh=_mesh(),
        compiler_params=pltpu.CompilerParams(needs_layout_passes=False),
        scratch_types=[
            pltpu.VMEM((NW * NW + 16,), jnp.int32),
            pltpu.VMEM((DPO + 16,), jnp.int32),
            pltpu.VMEM((CCAP,), jnp.int32),
            pltpu.VMEM((CCAP,), jnp.float32),
            pltpu.VMEM((CCAP,), jnp.int32),
            pltpu.VMEM((CCAP,), jnp.int32),
            pltpu.VMEM((GPAD,), jnp.float32),
            pltpu.VMEM((GPAD,), jnp.int32),
            pltpu.VMEM((GPAD,), jnp.int32),
            pltpu.VMEM((GPAD,), jnp.float32),
            pltpu.VMEM((336,), jnp.int32),
            pltpu.VMEM((336,), jnp.int32),
            pltpu.VMEM((64,), jnp.int32),
            pltpu.VMEM((16,), jnp.int32),
            pltpu.VMEM((16,), jnp.float32),
        ],
    )


def kernel(inputs, A, emb, W1, b1, W2, b2, ggnn_w, w_ih, w_hh, b_ih, b_hh):
    hidden = emb
    src, dst = A[0], A[1]

    indeg = _sc_degree(dst)
    deg = 1.0 + indeg
    dinv = 1.0 / jnp.sqrt(deg)
    dcol = dinv[:, None]

    xw1 = _pallas_matmul(hidden, W1)
    agg1 = _sc_aggregate(src, dst, _pad_rows(dcol * xw1))
    h = jax.nn.relu(dcol * agg1 + dcol * dcol * xw1 + b1)

    xw2 = _pallas_matmul(h, W2)
    agg2 = _sc_aggregate(src, dst, _pad_rows(dcol * xw2))
    h = jax.nn.relu(dcol * agg2 + dcol * dcol * xw2 + b2)

    nrm = jnp.maximum(jnp.linalg.norm(h, axis=1), 1e-8)
    hn = h / nrm[:, None]
    counts, dstc, scc, srcc, idxc = _t1_kernel()(src, dst, hn)
    deg_i = jnp.zeros((NP,), jnp.int32).at[:N].set(indeg.astype(jnp.int32))
    maskc = _t2_kernel()(counts, dstc, scc, srcc, idxc, deg_i)
    mask = maskc[:E] > 0.5

    src_m = jnp.where(mask, src, N + (jnp.arange(E, dtype=jnp.int32) & (PAD - 1)))

    x = hidden
    for i in range(L_GGNN):
        m = _pallas_matmul(x, ggnn_w[i])
        agg = _sc_aggregate(src_m, dst, _pad_rows(m))
        gi = _pallas_matmul(agg, w_ih.T) + b_ih
        gh = _pallas_matmul(x, w_hh.T) + b_hh
        i_r, i_z, i_n = jnp.split(gi, 3, axis=1)
        h_r, h_z, h_n = jnp.split(gh, 3, axis=1)
        r = jax.nn.sigmoid(i_r + h_r)
        z = jax.nn.sigmoid(i_z + h_z)
        nn_ = jnp.tanh(i_n + r * h_n)
        x = (1.0 - z) * nn_ + z * x
    return x
